# baseline (speedup 1.0000x reference)
import functools

import jax
import jax.numpy as jnp
from jax import lax
from jax.experimental import pallas as pl
from jax.experimental.pallas import tpu as pltpu
from jax.experimental.pallas import tpu_sc as plsc

NUM_USER = 10000
NUM_ITEM = 40000
N = NUM_USER + NUM_ITEM
E = 800000
E2 = 2 * E
NEG_GAT = 0.2
NEG = 0.01

NC = 2
NS = 16
L = 16

C1 = 640
C2 = 640
NJ = C1 // 128
NJW = C2 // 128
ROWS_T = N // NS


def _leaky(x, s):
    return jnp.where(x >= 0, x, s * x)


def _norm_rows(y):
    return y / jnp.maximum(jnp.sqrt(jnp.sum(y * y, axis=-1, keepdims=True)), 1e-12)


BR = 2000
NB = N // BR
UB = NUM_USER // BR


def _prologue_body(vf_ref, pref_ref, wm_ref, bm_ref, wg_ref, asrc_ref, adst_ref,
                   ide_ref, x_ref, t0_ref, t1_ref, t2_ref, t3_ref, s_ref, t_ref):
    i = pl.program_id(0)
    feat = _leaky(jnp.dot(vf_ref[...], wm_ref[...],
                          preferred_element_type=jnp.float32) + bm_ref[...], NEG)
    feat = _norm_rows(feat)
    prefn = _norm_rows(pref_ref[...])
    xblk = jnp.where(i < UB, prefn, feat)
    x_ref[...] = xblk
    xp = jnp.dot(xblk, wg_ref[...], preferred_element_type=jnp.float32)
    t0_ref[...] = xp[:, :32]
    t1_ref[...] = xp[:, 32:]
    xi = _norm_rows(ide_ref[...])
    t2_ref[...] = xi[:, :32]
    t3_ref[...] = xi[:, 32:]
    s_ref[0, 0, :] = jnp.dot(xp, asrc_ref[...], preferred_element_type=jnp.float32)
    t_ref[0, 0, :] = jnp.dot(xp, adst_ref[...], preferred_element_type=jnp.float32)


def _prologue(v_feat, preference, W_mlp, b_mlp, W_gat, a_src, a_dst, id_embedding):
    f32 = jnp.float32
    return pl.pallas_call(
        _prologue_body,
        grid=(NB,),
        in_specs=[
            pl.BlockSpec((BR, 128), lambda i: (jnp.maximum(i - UB, 0), 0)),
            pl.BlockSpec((BR, 64), lambda i: (jnp.minimum(i, UB - 1), 0)),
            pl.BlockSpec((128, 64), lambda i: (0, 0)),
            pl.BlockSpec((64,), lambda i: (0,)),
            pl.BlockSpec((64, 64), lambda i: (0, 0)),
            pl.BlockSpec((64,), lambda i: (0,)),
            pl.BlockSpec((64,), lambda i: (0,)),
            pl.BlockSpec((BR, 64), lambda i: (i, 0)),
        ],
        out_specs=[
            pl.BlockSpec((BR, 64), lambda i: (i, 0)),
            pl.BlockSpec((BR, 32), lambda i: (i, 0)),
            pl.BlockSpec((BR, 32), lambda i: (i, 0)),
            pl.BlockSpec((BR, 32), lambda i: (i, 0)),
            pl.BlockSpec((BR, 32), lambda i: (i, 0)),
            pl.BlockSpec((1, 1, BR), lambda i: (i, 0, 0)),
            pl.BlockSpec((1, 1, BR), lambda i: (i, 0, 0)),
        ],
        out_shape=[
            jax.ShapeDtypeStruct((N, 64), f32),
            jax.ShapeDtypeStruct((N, 32), f32),
            jax.ShapeDtypeStruct((N, 32), f32),
            jax.ShapeDtypeStruct((N, 32), f32),
            jax.ShapeDtypeStruct((N, 32), f32),
            jax.ShapeDtypeStruct((NB, 1, BR), f32),
            jax.ShapeDtypeStruct((NB, 1, BR), f32),
        ],
    )(v_feat, preference, W_mlp.T, b_mlp, W_gat.T, a_src, a_dst, id_embedding)


def _full16(v):
    return jnp.full((L,), v, jnp.int32)


_IOTA = lambda: lax.iota(jnp.int32, L)


def _sc_mesh():
    return plsc.VectorSubcoreMesh(core_axis_name="c", subcore_axis_name="s",
                                  num_cores=NC, num_subcores=NS)


def _weights_body(src3d, dst3d, s_hbm, t_hbm, z1_hbm,
                  w_hbm, den0_hbm, den1_hbm,
                  s_v, t_v, srcb, dstb, wfb, wbb, den_sh, sem):
    c = lax.axis_index("c")
    sid = lax.axis_index("s")
    wid = c * NS + sid
    pltpu.sync_copy(s_hbm, s_v)
    pltpu.sync_copy(t_hbm, t_v)

    @pl.when(sid == 0)
    def _():
        pltpu.sync_copy(z1_hbm, den_sh)
    plsc.subcore_barrier()

    n_chunks = E // C1
    trips = (n_chunks // (NC * NS)) + jnp.where(wid < (n_chunks % (NC * NS)), 1, 0)

    def chunk(ci, _):
        chunk_id = wid + ci * (NC * NS)
        base = chunk_id * C1
        pltpu.sync_copy(src3d.at[chunk_id], srcb)
        pltpu.sync_copy(dst3d.at[chunk_id], dstb)

        for j in range(NJ):
            def grp(g, _, j=j):
                si = srcb[j, pl.ds(g * L, L)]
                di = dstb[j, pl.ds(g * L, L)]
                ss = plsc.load_gather(s_v, [si])
                td = plsc.load_gather(t_v, [di])
                sd = plsc.load_gather(s_v, [di])
                ts = plsc.load_gather(t_v, [si])
                ef = ss + td
                eb = sd + ts
                wf = jnp.exp(jnp.where(ef >= 0, ef, NEG_GAT * ef))
                wb = jnp.exp(jnp.where(eb >= 0, eb, NEG_GAT * eb))
                wfb[pl.ds(j * 128 + g * L, L)] = wf
                wbb[pl.ds(j * 128 + g * L, L)] = wb
                return _

            lax.fori_loop(0, 128 // L, grp, None, unroll=4)
        for j in range(NJ):
            pltpu.async_copy(wfb.at[pl.ds(j * 128, 128)],
                             den_sh.at[dstb.at[j]], sem, add=True).wait()
            pltpu.async_copy(wbb.at[pl.ds(j * 128, 128)],
                             den_sh.at[srcb.at[j]], sem, add=True).wait()
        pltpu.sync_copy(wfb, w_hbm.at[pl.ds(base, C1)])
        pltpu.sync_copy(wbb, w_hbm.at[pl.ds(E + base, C1)])
        return _

    lax.fori_loop(0, trips, chunk, None)
    plsc.subcore_barrier()

    @pl.when((sid == 0) & (c == 0))
    def _():
        pltpu.sync_copy(den_sh, den0_hbm)

    @pl.when((sid == 0) & (c == 1))
    def _():
        pltpu.sync_copy(den_sh, den1_hbm)


def _sc_weights(src3d, dst3d, s, t, z1):
    f32 = jnp.float32
    i32 = jnp.int32
    fn = pl.kernel(
        _weights_body,
        out_type=(jax.ShapeDtypeStruct((E2,), f32),
                  jax.ShapeDtypeStruct((N,), f32),
                  jax.ShapeDtypeStruct((N,), f32)),
        mesh=_sc_mesh(),
        compiler_params=pltpu.CompilerParams(needs_layout_passes=False, use_tc_tiling_on_sc=False),
        scratch_types=[
            pltpu.VMEM((N,), f32),
            pltpu.VMEM((N,), f32),
            pltpu.VMEM((NJ, 128), i32),
            pltpu.VMEM((NJ, 128), i32),
            pltpu.VMEM((C1,), f32),
            pltpu.VMEM((C1,), f32),
            pltpu.VMEM_SHARED((N,), f32),
            pltpu.SemaphoreType.DMA,
        ],
    )
    return fn(src3d, dst3d, s, t, z1)


ROWS_A = 3128
ROWS_TAIL = N - 15 * ROWS_A


def _make_agg_body(K):
    def body(*refs):
        tgt3d, sos3d, w_hbm = refs[0], refs[1], refs[2]
        tabs = refs[3:3 + K]
        z_hbm = refs[3 + K]
        outs = refs[4 + K:4 + 2 * K]
        tb, ob, wbuf, rows, acc, sem, sem2 = refs[4 + 2 * K:]

        c = lax.axis_index("c")
        sid = lax.axis_index("s")
        n_chunks = E // C2
        trips = (n_chunks // NS) + jnp.where(sid < (n_chunks % NS), 1, 0)

        def tile_slab(arr2d, off):
            return None

        for k in range(K):

            @pl.when(sid < 15)
            def _():
                pltpu.sync_copy(z_hbm.at[pl.ds(sid * ROWS_A, ROWS_A)],
                                acc.at[pl.ds(sid * ROWS_A, ROWS_A)])

            @pl.when(sid == 15)
            def _():
                pltpu.sync_copy(z_hbm.at[pl.ds(15 * ROWS_A, ROWS_TAIL)],
                                acc.at[pl.ds(15 * ROWS_A, ROWS_TAIL)])
            plsc.subcore_barrier()

            def chunk(ci, _):
                chunk_id = sid + ci * NS
                chunk_global = c * n_chunks + chunk_id
                base = chunk_global * C2
                pltpu.sync_copy(tgt3d.at[chunk_global], tb)
                pltpu.sync_copy(sos3d.at[chunk_global], ob)
                pltpu.sync_copy(w_hbm.at[pl.ds(base, C2)], wbuf)
                gets = [pltpu.async_copy(tabs[k].at[ob.at[j]],
                                         rows.at[pl.ds(j * 128, 128)], sem)
                        for j in range(NJW)]
                puts = []
                for j in range(NJW):
                    gets[j].wait()

                    @plsc.parallel_loop(j * 128, (j + 1) * 128, step=1, unroll=8)
                    def srow(r, j=j):
                        wspl = plsc.load_gather(wbuf, [_full16(r)])
                        for h in range(2):
                            cidx = _IOTA() + h * L
                            v = plsc.load_gather(rows, [_full16(r), cidx])
                            plsc.store_scatter(rows, [_full16(r), cidx], v * wspl)
                    puts.append(pltpu.async_copy(rows.at[pl.ds(j * 128, 128)],
                                                 acc.at[tb.at[j]], sem2, add=True))
                for p in puts:
                    p.wait()
                return _

            lax.fori_loop(0, trips, chunk, None)
            plsc.subcore_barrier()

            @pl.when(sid < 15)
            def _():
                pltpu.sync_copy(acc.at[pl.ds(sid * ROWS_A, ROWS_A)],
                                outs[k].at[pl.ds(c * N + sid * ROWS_A, ROWS_A)])

            @pl.when(sid == 15)
            def _():
                pltpu.sync_copy(acc.at[pl.ds(15 * ROWS_A, ROWS_TAIL)],
                                outs[k].at[pl.ds(c * N + 15 * ROWS_A, ROWS_TAIL)])
            plsc.subcore_barrier()

    return body


def _sc_aggregate(tgt3d, sos3d, w, tabs, z):
    f32 = jnp.float32
    i32 = jnp.int32
    K = len(tabs)
    fn = pl.kernel(
        _make_agg_body(K),
        out_type=tuple(jax.ShapeDtypeStruct((NC * N, 32), f32) for _ in range(K)),
        mesh=_sc_mesh(),
        compiler_params=pltpu.CompilerParams(needs_layout_passes=False, use_tc_tiling_on_sc=False),
        scratch_types=[
            pltpu.VMEM((NJW, 128), i32),
            pltpu.VMEM((NJW, 128), i32),
            pltpu.VMEM((C2,), f32),
            pltpu.VMEM((C2, 32), f32),
            pltpu.VMEM_SHARED((N, 32), f32),
            pltpu.SemaphoreType.DMA,
            pltpu.SemaphoreType.DMA,
        ],
    )
    return fn(tgt3d, sos3d, w, *tabs, z)


def _mid(outsA, den_part, x, xia, xib, Wl1, Wr1, b1, b_gat):
    f32 = jnp.float32
    blk32 = lambda i: (i, 0)
    half_specs = []
    args = []
    for o in outsA:
        args.extend([o, o])
        half_specs.extend([
            pl.BlockSpec((BR, 32), lambda i: (i, 0)),
            pl.BlockSpec((BR, 32), lambda i: (i + NB, 0)),
        ])
    dp4 = den_part.reshape(NC, NB, 1, BR)
    args.extend([dp4, dp4])
    half_specs.extend([
        pl.BlockSpec((1, 1, 1, BR), lambda i: (0, i, 0, 0)),
        pl.BlockSpec((1, 1, 1, BR), lambda i: (1, i, 0, 0)),
    ])
    return pl.pallas_call(
        _mid_body2,
        grid=(NB,),
        in_specs=half_specs + [
            pl.BlockSpec((BR, 64), blk32),
            pl.BlockSpec((BR, 32), blk32),
            pl.BlockSpec((BR, 32), blk32),
            pl.BlockSpec((64, 64), lambda i: (0, 0)),
            pl.BlockSpec((64, 64), lambda i: (0, 0)),
            pl.BlockSpec((64,), lambda i: (0,)),
            pl.BlockSpec((64,), lambda i: (0,)),
        ],
        out_specs=[
            pl.BlockSpec((BR, 64), blk32),
            pl.BlockSpec((BR, 64), blk32),
            pl.BlockSpec((BR, 32), blk32),
            pl.BlockSpec((BR, 32), blk32),
        ],
        out_shape=[
            jax.ShapeDtypeStruct((N, 64), f32),
            jax.ShapeDtypeStruct((N, 64), f32),
            jax.ShapeDtypeStruct((N, 32), f32),
            jax.ShapeDtypeStruct((N, 32), f32),
        ],
    )(*args, x, xia, xib, Wl1.T, Wr1.T, b1, b_gat)


def _mid_body2(o0a, o0b, o1a, o1b, o2a, o2b, o3a, o3b, dpa, dpb,
               x_ref, xia, xib, wl1, wr1, b1_ref, bg_ref,
               vrep_ref, x1_ref, x1a_ref, x1b_ref):
    den = dpa[0, 0, 0, :] + dpb[0, 0, 0, :] + 1e-16
    inv = (1.0 / den)[:, None]
    agg_xp = jnp.concatenate([o0a[...] + o0b[...], o1a[...] + o1b[...]], axis=1)
    xh = _leaky(agg_xp * inv + bg_ref[...], NEG)
    vrep_ref[...] = x_ref[...] + xh
    agg_xi = jnp.concatenate([o2a[...] + o2b[...], o3a[...] + o3b[...]], axis=1)
    xi = jnp.concatenate([xia[...], xib[...]], axis=1)
    x1 = _leaky(jnp.dot(agg_xi * inv, wl1[...], preferred_element_type=jnp.float32)
                + jnp.dot(xi, wr1[...], preferred_element_type=jnp.float32)
                + b1_ref[...], NEG)
    x1_ref[...] = x1
    x1a_ref[...] = x1[:, :32]
    x1b_ref[...] = x1[:, 32:]


def _final_body(b0a, b0b, b1a, b1b, dpa, dpb, x1_ref, xia, xib, vrep_ref,
                wl2, wr2, b2_ref, out_ref):
    den = dpa[0, 0, 0, :] + dpb[0, 0, 0, :] + 1e-16
    inv = (1.0 / den)[:, None]
    agg2 = jnp.concatenate([b0a[...] + b0b[...], b1a[...] + b1b[...]], axis=1)
    x1 = x1_ref[...]
    x2 = _leaky(jnp.dot(agg2 * inv, wl2[...], preferred_element_type=jnp.float32)
                + jnp.dot(x1, wr2[...], preferred_element_type=jnp.float32)
                + b2_ref[...], NEG)
    xi = jnp.concatenate([xia[...], xib[...]], axis=1)
    out_ref[:, :64] = xi + x1 + x2
    out_ref[:, 64:] = vrep_ref[...]


def _final(outsB, den_part, x1, xia, xib, vrep, Wl2, Wr2, b2):
    f32 = jnp.float32
    blk32 = lambda i: (i, 0)
    half_specs = []
    args = []
    for o in outsB:
        args.extend([o, o])
        half_specs.extend([
            pl.BlockSpec((BR, 32), lambda i: (i, 0)),
            pl.BlockSpec((BR, 32), lambda i: (i + NB, 0)),
        ])
    dp4 = den_part.reshape(NC, NB, 1, BR)
    args.extend([dp4, dp4])
    half_specs.extend([
        pl.BlockSpec((1, 1, 1, BR), lambda i: (0, i, 0, 0)),
        pl.BlockSpec((1, 1, 1, BR), lambda i: (1, i, 0, 0)),
    ])
    return pl.pallas_call(
        _final_body,
        grid=(NB,),
        in_specs=half_specs + [
            pl.BlockSpec((BR, 64), blk32),
            pl.BlockSpec((BR, 32), blk32),
            pl.BlockSpec((BR, 32), blk32),
            pl.BlockSpec((BR, 64), blk32),
            pl.BlockSpec((64, 64), lambda i: (0, 0)),
            pl.BlockSpec((64, 64), lambda i: (0, 0)),
            pl.BlockSpec((64,), lambda i: (0,)),
        ],
        out_specs=pl.BlockSpec((BR, 128), blk32),
        out_shape=jax.ShapeDtypeStruct((N, 128), f32),
    )(*args, x1, xia, xib, vrep, Wl2.T, Wr2.T, b2)


def kernel(edge_index, v_feat, preference, W_mlp, b_mlp, W_gat, a_src, a_dst,
           b_gat, id_embedding, Wl1, Wr1, b1, Wl2, Wr2, b2):
    src = edge_index[0].astype(jnp.int32)
    dst = edge_index[1].astype(jnp.int32)

    x, t0, t1, t2, t3, s, t = _prologue(
        v_feat, preference, W_mlp, b_mlp, W_gat, a_src, a_dst, id_embedding)

    src3d = src.reshape(E // C1, NJ, 128)
    dst3d = dst.reshape(E // C1, NJ, 128)
    tgt3d = jnp.concatenate([dst, src]).reshape(E2 // C2, NJW, 128)
    sos3d = jnp.concatenate([src, dst]).reshape(E2 // C2, NJW, 128)
    z1 = jnp.zeros((N,), jnp.float32)
    z32 = jnp.zeros((N, 32), jnp.float32)

    s_flat = s.reshape(N)
    t_flat = t.reshape(N)
    w, den0, den1 = _sc_weights(src3d, dst3d, s_flat, t_flat, z1)
    den_part = jnp.stack([den0, den1])
    import os as _os
    if _os.environ.get("_SC_BISECT") == "1":
        return w, den_part
    outsA = _sc_aggregate(tgt3d, sos3d, w, [t0, t1, t2, t3], z32)
    vrep, x1, x1a, x1b = _mid(outsA, den_part, x, t2, t3, Wl1, Wr1, b1, b_gat)
    outsB = _sc_aggregate(tgt3d, sos3d, w, [x1a, x1b], z32)
    return _final(outsB, den_part, x1, t2, t3, vrep, Wl2, Wr2, b2)


# speedup vs baseline: 1.2255x; 1.2255x over previous
import functools

import jax
import jax.numpy as jnp
from jax import lax
from jax.experimental import pallas as pl
from jax.experimental.pallas import tpu as pltpu
from jax.experimental.pallas import tpu_sc as plsc

NUM_USER = 10000
NUM_ITEM = 40000
N = NUM_USER + NUM_ITEM
E = 800000
E2 = 2 * E
NEG_GAT = 0.2
NEG = 0.01

NC = 2
NS = 16
L = 16

C1 = 640
C2 = 640
NJ = C1 // 128
NJW = C2 // 128
ROWS_T = N // NS


def _leaky(x, s):
    return jnp.where(x >= 0, x, s * x)


def _norm_rows(y):
    return y / jnp.maximum(jnp.sqrt(jnp.sum(y * y, axis=-1, keepdims=True)), 1e-12)


BR = 2000
NB = N // BR
UB = NUM_USER // BR


def _prologue_body(vf_ref, pref_ref, wm_ref, bm_ref, wg_ref, asrc_ref, adst_ref,
                   ide_ref, x_ref, t0_ref, t1_ref, t2_ref, t3_ref, s_ref, t_ref):
    i = pl.program_id(0)
    feat = _leaky(jnp.dot(vf_ref[...], wm_ref[...],
                          preferred_element_type=jnp.float32) + bm_ref[...], NEG)
    feat = _norm_rows(feat)
    prefn = _norm_rows(pref_ref[...])
    xblk = jnp.where(i < UB, prefn, feat)
    x_ref[...] = xblk
    xp = jnp.dot(xblk, wg_ref[...], preferred_element_type=jnp.float32)
    t0_ref[...] = xp[:, :32]
    t1_ref[...] = xp[:, 32:]
    xi = _norm_rows(ide_ref[...])
    t2_ref[...] = xi[:, :32]
    t3_ref[...] = xi[:, 32:]
    s_ref[0, 0, :] = jnp.dot(xp, asrc_ref[...], preferred_element_type=jnp.float32)
    t_ref[0, 0, :] = jnp.dot(xp, adst_ref[...], preferred_element_type=jnp.float32)


def _prologue(v_feat, preference, W_mlp, b_mlp, W_gat, a_src, a_dst, id_embedding):
    f32 = jnp.float32
    return pl.pallas_call(
        _prologue_body,
        grid=(NB,),
        in_specs=[
            pl.BlockSpec((BR, 128), lambda i: (jnp.maximum(i - UB, 0), 0)),
            pl.BlockSpec((BR, 64), lambda i: (jnp.minimum(i, UB - 1), 0)),
            pl.BlockSpec((128, 64), lambda i: (0, 0)),
            pl.BlockSpec((64,), lambda i: (0,)),
            pl.BlockSpec((64, 64), lambda i: (0, 0)),
            pl.BlockSpec((64,), lambda i: (0,)),
            pl.BlockSpec((64,), lambda i: (0,)),
            pl.BlockSpec((BR, 64), lambda i: (i, 0)),
        ],
        out_specs=[
            pl.BlockSpec((BR, 64), lambda i: (i, 0)),
            pl.BlockSpec((BR, 32), lambda i: (i, 0)),
            pl.BlockSpec((BR, 32), lambda i: (i, 0)),
            pl.BlockSpec((BR, 32), lambda i: (i, 0)),
            pl.BlockSpec((BR, 32), lambda i: (i, 0)),
            pl.BlockSpec((1, 1, BR), lambda i: (i, 0, 0)),
            pl.BlockSpec((1, 1, BR), lambda i: (i, 0, 0)),
        ],
        out_shape=[
            jax.ShapeDtypeStruct((N, 64), f32),
            jax.ShapeDtypeStruct((N, 32), f32),
            jax.ShapeDtypeStruct((N, 32), f32),
            jax.ShapeDtypeStruct((N, 32), f32),
            jax.ShapeDtypeStruct((N, 32), f32),
            jax.ShapeDtypeStruct((NB, 1, BR), f32),
            jax.ShapeDtypeStruct((NB, 1, BR), f32),
        ],
    )(v_feat, preference, W_mlp.T, b_mlp, W_gat.T, a_src, a_dst, id_embedding)


def _full16(v):
    return jnp.full((L,), v, jnp.int32)


_IOTA = lambda: lax.iota(jnp.int32, L)


def _sc_mesh():
    return plsc.VectorSubcoreMesh(core_axis_name="c", subcore_axis_name="s",
                                  num_cores=NC, num_subcores=NS)


def _weights_body(src3d, dst3d, s_hbm, t_hbm, z1_hbm,
                  w_hbm, den0_hbm, den1_hbm,
                  s_v, t_v, srcb, dstb, wfb, wbb, den_sh, sem):
    c = lax.axis_index("c")
    sid = lax.axis_index("s")
    wid = c * NS + sid
    pltpu.sync_copy(s_hbm, s_v)
    pltpu.sync_copy(t_hbm, t_v)

    @pl.when(sid == 0)
    def _():
        pltpu.sync_copy(z1_hbm, den_sh)
    plsc.subcore_barrier()

    n_chunks = E // C1
    trips = (n_chunks // (NC * NS)) + jnp.where(wid < (n_chunks % (NC * NS)), 1, 0)

    def chunk(ci, _):
        chunk_id = wid + ci * (NC * NS)
        base = chunk_id * C1
        pltpu.sync_copy(src3d.at[chunk_id], srcb)
        pltpu.sync_copy(dst3d.at[chunk_id], dstb)

        for j in range(NJ):
            def grp(g, _, j=j):
                si = srcb[j, pl.ds(g * L, L)]
                di = dstb[j, pl.ds(g * L, L)]
                ss = plsc.load_gather(s_v, [si])
                td = plsc.load_gather(t_v, [di])
                sd = plsc.load_gather(s_v, [di])
                ts = plsc.load_gather(t_v, [si])
                ef = ss + td
                eb = sd + ts
                wf = jnp.exp(jnp.where(ef >= 0, ef, NEG_GAT * ef))
                wb = jnp.exp(jnp.where(eb >= 0, eb, NEG_GAT * eb))
                wfb[pl.ds(j * 128 + g * L, L)] = wf
                wbb[pl.ds(j * 128 + g * L, L)] = wb
                return _

            lax.fori_loop(0, 128 // L, grp, None, unroll=4)
        for j in range(NJ):
            pltpu.async_copy(wfb.at[pl.ds(j * 128, 128)],
                             den_sh.at[dstb.at[j]], sem, add=True).wait()
            pltpu.async_copy(wbb.at[pl.ds(j * 128, 128)],
                             den_sh.at[srcb.at[j]], sem, add=True).wait()
        pltpu.sync_copy(wfb, w_hbm.at[pl.ds(base, C1)])
        pltpu.sync_copy(wbb, w_hbm.at[pl.ds(E + base, C1)])
        return _

    lax.fori_loop(0, trips, chunk, None)
    plsc.subcore_barrier()

    @pl.when((sid == 0) & (c == 0))
    def _():
        pltpu.sync_copy(den_sh, den0_hbm)

    @pl.when((sid == 0) & (c == 1))
    def _():
        pltpu.sync_copy(den_sh, den1_hbm)


def _sc_weights(src3d, dst3d, s, t, z1):
    f32 = jnp.float32
    i32 = jnp.int32
    fn = pl.kernel(
        _weights_body,
        out_type=(jax.ShapeDtypeStruct((E2,), f32),
                  jax.ShapeDtypeStruct((N,), f32),
                  jax.ShapeDtypeStruct((N,), f32)),
        mesh=_sc_mesh(),
        compiler_params=pltpu.CompilerParams(needs_layout_passes=False, use_tc_tiling_on_sc=False),
        scratch_types=[
            pltpu.VMEM((N,), f32),
            pltpu.VMEM((N,), f32),
            pltpu.VMEM((NJ, 128), i32),
            pltpu.VMEM((NJ, 128), i32),
            pltpu.VMEM((C1,), f32),
            pltpu.VMEM((C1,), f32),
            pltpu.VMEM_SHARED((N,), f32),
            pltpu.SemaphoreType.DMA,
        ],
    )
    return fn(src3d, dst3d, s, t, z1)


ROWS_A = 3128
ROWS_TAIL = N - 15 * ROWS_A


def _make_agg_body(K):
    def body(*refs):
        tgt3d, sos3d, w_hbm = refs[0], refs[1], refs[2]
        tabs = refs[3:3 + K]
        z_hbm = refs[3 + K]
        outs = refs[4 + K:4 + 2 * K]
        tb, ob, wbuf, rows, acc, sem, sem2, sem3 = refs[4 + 2 * K:]

        c = lax.axis_index("c")
        sid = lax.axis_index("s")
        n_chunks = E // C2
        trips = (n_chunks // NS) + jnp.where(sid < (n_chunks % NS), 1, 0)

        def tile_slab(arr2d, off):
            return None

        for k in range(K):

            @pl.when(sid < 15)
            def _():
                pltpu.sync_copy(z_hbm.at[pl.ds(sid * ROWS_A, ROWS_A)],
                                acc.at[pl.ds(sid * ROWS_A, ROWS_A)])

            @pl.when(sid == 15)
            def _():
                pltpu.sync_copy(z_hbm.at[pl.ds(15 * ROWS_A, ROWS_TAIL)],
                                acc.at[pl.ds(15 * ROWS_A, ROWS_TAIL)])
            plsc.subcore_barrier()

            def chunk(ci, _):
                chunk_id = sid + ci * NS
                chunk_global = c * n_chunks + chunk_id
                base = chunk_global * C2
                i_ob = pltpu.async_copy(sos3d.at[chunk_global], ob, sem3)
                i_tb = pltpu.async_copy(tgt3d.at[chunk_global], tb, sem3)
                i_wb = pltpu.async_copy(w_hbm.at[pl.ds(base, C2)], wbuf, sem3)
                i_ob.wait()
                gets = [pltpu.async_copy(tabs[k].at[ob.at[j]],
                                         rows.at[pl.ds(j * 128, 128)], sem)
                        for j in range(NJW)]
                i_wb.wait()
                i_tb.wait()
                puts = []
                for j in range(NJW):
                    gets[j].wait()

                    @plsc.parallel_loop(j * 128, (j + 1) * 128, step=1, unroll=8)
                    def srow(r, j=j):
                        wspl = plsc.load_gather(wbuf, [_full16(r)])
                        for h in range(2):
                            cidx = _IOTA() + h * L
                            v = plsc.load_gather(rows, [_full16(r), cidx])
                            plsc.store_scatter(rows, [_full16(r), cidx], v * wspl)
                    puts.append(pltpu.async_copy(rows.at[pl.ds(j * 128, 128)],
                                                 acc.at[tb.at[j]], sem2, add=True))
                for p in puts:
                    p.wait()
                return _

            lax.fori_loop(0, trips, chunk, None)
            plsc.subcore_barrier()

            @pl.when(sid < 15)
            def _():
                pltpu.sync_copy(acc.at[pl.ds(sid * ROWS_A, ROWS_A)],
                                outs[k].at[pl.ds(c * N + sid * ROWS_A, ROWS_A)])

            @pl.when(sid == 15)
            def _():
                pltpu.sync_copy(acc.at[pl.ds(15 * ROWS_A, ROWS_TAIL)],
                                outs[k].at[pl.ds(c * N + 15 * ROWS_A, ROWS_TAIL)])
            plsc.subcore_barrier()

    return body


def _sc_aggregate(tgt3d, sos3d, w, tabs, z):
    f32 = jnp.float32
    i32 = jnp.int32
    K = len(tabs)
    fn = pl.kernel(
        _make_agg_body(K),
        out_type=tuple(jax.ShapeDtypeStruct((NC * N, 32), f32) for _ in range(K)),
        mesh=_sc_mesh(),
        compiler_params=pltpu.CompilerParams(needs_layout_passes=False, use_tc_tiling_on_sc=False),
        scratch_types=[
            pltpu.VMEM((NJW, 128), i32),
            pltpu.VMEM((NJW, 128), i32),
            pltpu.VMEM((C2,), f32),
            pltpu.VMEM((C2, 32), f32),
            pltpu.VMEM_SHARED((N, 32), f32),
            pltpu.SemaphoreType.DMA,
            pltpu.SemaphoreType.DMA,
            pltpu.SemaphoreType.DMA,
        ],
    )
    return fn(tgt3d, sos3d, w, *tabs, z)


def _mid(outsA, den_part, x, xia, xib, Wl1, Wr1, b1, b_gat):
    f32 = jnp.float32
    blk32 = lambda i: (i, 0)
    half_specs = []
    args = []
    for o in outsA:
        args.extend([o, o])
        half_specs.extend([
            pl.BlockSpec((BR, 32), lambda i: (i, 0)),
            pl.BlockSpec((BR, 32), lambda i: (i + NB, 0)),
        ])
    dp4 = den_part.reshape(NC, NB, 1, BR)
    args.extend([dp4, dp4])
    half_specs.extend([
        pl.BlockSpec((1, 1, 1, BR), lambda i: (0, i, 0, 0)),
        pl.BlockSpec((1, 1, 1, BR), lambda i: (1, i, 0, 0)),
    ])
    return pl.pallas_call(
        _mid_body2,
        grid=(NB,),
        in_specs=half_specs + [
            pl.BlockSpec((BR, 64), blk32),
            pl.BlockSpec((BR, 32), blk32),
            pl.BlockSpec((BR, 32), blk32),
            pl.BlockSpec((64, 64), lambda i: (0, 0)),
            pl.BlockSpec((64, 64), lambda i: (0, 0)),
            pl.BlockSpec((64,), lambda i: (0,)),
            pl.BlockSpec((64,), lambda i: (0,)),
        ],
        out_specs=[
            pl.BlockSpec((BR, 64), blk32),
            pl.BlockSpec((BR, 64), blk32),
            pl.BlockSpec((BR, 32), blk32),
            pl.BlockSpec((BR, 32), blk32),
        ],
        out_shape=[
            jax.ShapeDtypeStruct((N, 64), f32),
            jax.ShapeDtypeStruct((N, 64), f32),
            jax.ShapeDtypeStruct((N, 32), f32),
            jax.ShapeDtypeStruct((N, 32), f32),
        ],
    )(*args, x, xia, xib, Wl1.T, Wr1.T, b1, b_gat)


def _mid_body2(o0a, o0b, o1a, o1b, o2a, o2b, o3a, o3b, dpa, dpb,
               x_ref, xia, xib, wl1, wr1, b1_ref, bg_ref,
               vrep_ref, x1_ref, x1a_ref, x1b_ref):
    den = dpa[0, 0, 0, :] + dpb[0, 0, 0, :] + 1e-16
    inv = (1.0 / den)[:, None]
    agg_xp = jnp.concatenate([o0a[...] + o0b[...], o1a[...] + o1b[...]], axis=1)
    xh = _leaky(agg_xp * inv + bg_ref[...], NEG)
    vrep_ref[...] = x_ref[...] + xh
    agg_xi = jnp.concatenate([o2a[...] + o2b[...], o3a[...] + o3b[...]], axis=1)
    xi = jnp.concatenate([xia[...], xib[...]], axis=1)
    x1 = _leaky(jnp.dot(agg_xi * inv, wl1[...], preferred_element_type=jnp.float32)
                + jnp.dot(xi, wr1[...], preferred_element_type=jnp.float32)
                + b1_ref[...], NEG)
    x1_ref[...] = x1
    x1a_ref[...] = x1[:, :32]
    x1b_ref[...] = x1[:, 32:]


def _final_body(b0a, b0b, b1a, b1b, dpa, dpb, x1_ref, xia, xib, vrep_ref,
                wl2, wr2, b2_ref, out_ref):
    den = dpa[0, 0, 0, :] + dpb[0, 0, 0, :] + 1e-16
    inv = (1.0 / den)[:, None]
    agg2 = jnp.concatenate([b0a[...] + b0b[...], b1a[...] + b1b[...]], axis=1)
    x1 = x1_ref[...]
    x2 = _leaky(jnp.dot(agg2 * inv, wl2[...], preferred_element_type=jnp.float32)
                + jnp.dot(x1, wr2[...], preferred_element_type=jnp.float32)
                + b2_ref[...], NEG)
    xi = jnp.concatenate([xia[...], xib[...]], axis=1)
    out_ref[:, :64] = xi + x1 + x2
    out_ref[:, 64:] = vrep_ref[...]


def _final(outsB, den_part, x1, xia, xib, vrep, Wl2, Wr2, b2):
    f32 = jnp.float32
    blk32 = lambda i: (i, 0)
    half_specs = []
    args = []
    for o in outsB:
        args.extend([o, o])
        half_specs.extend([
            pl.BlockSpec((BR, 32), lambda i: (i, 0)),
            pl.BlockSpec((BR, 32), lambda i: (i + NB, 0)),
        ])
    dp4 = den_part.reshape(NC, NB, 1, BR)
    args.extend([dp4, dp4])
    half_specs.extend([
        pl.BlockSpec((1, 1, 1, BR), lambda i: (0, i, 0, 0)),
        pl.BlockSpec((1, 1, 1, BR), lambda i: (1, i, 0, 0)),
    ])
    return pl.pallas_call(
        _final_body,
        grid=(NB,),
        in_specs=half_specs + [
            pl.BlockSpec((BR, 64), blk32),
            pl.BlockSpec((BR, 32), blk32),
            pl.BlockSpec((BR, 32), blk32),
            pl.BlockSpec((BR, 64), blk32),
            pl.BlockSpec((64, 64), lambda i: (0, 0)),
            pl.BlockSpec((64, 64), lambda i: (0, 0)),
            pl.BlockSpec((64,), lambda i: (0,)),
        ],
        out_specs=pl.BlockSpec((BR, 128), blk32),
        out_shape=jax.ShapeDtypeStruct((N, 128), f32),
    )(*args, x1, xia, xib, vrep, Wl2.T, Wr2.T, b2)


def kernel(edge_index, v_feat, preference, W_mlp, b_mlp, W_gat, a_src, a_dst,
           b_gat, id_embedding, Wl1, Wr1, b1, Wl2, Wr2, b2):
    src = edge_index[0].astype(jnp.int32)
    dst = edge_index[1].astype(jnp.int32)

    x, t0, t1, t2, t3, s, t = _prologue(
        v_feat, preference, W_mlp, b_mlp, W_gat, a_src, a_dst, id_embedding)

    src3d = src.reshape(E // C1, NJ, 128)
    dst3d = dst.reshape(E // C1, NJ, 128)
    tgt3d = jnp.concatenate([dst, src]).reshape(E2 // C2, NJW, 128)
    sos3d = jnp.concatenate([src, dst]).reshape(E2 // C2, NJW, 128)
    z1 = jnp.zeros((N,), jnp.float32)
    z32 = jnp.zeros((N, 32), jnp.float32)

    s_flat = s.reshape(N)
    t_flat = t.reshape(N)
    w, den0, den1 = _sc_weights(src3d, dst3d, s_flat, t_flat, z1)
    den_part = jnp.stack([den0, den1])
    import os as _os
    if _os.environ.get("_SC_BISECT") == "1":
        return w, den_part
    outsA = _sc_aggregate(tgt3d, sos3d, w, [t0, t1, t2, t3], z32)
    vrep, x1, x1a, x1b = _mid(outsA, den_part, x, t2, t3, Wl1, Wr1, b1, b_gat)
    outsB = _sc_aggregate(tgt3d, sos3d, w, [x1a, x1b], z32)
    return _final(outsB, den_part, x1, t2, t3, vrep, Wl2, Wr2, b2)
